# Initial kernel scaffold; baseline (speedup 1.0000x reference)
import jax
import jax.numpy as jnp
from jax.experimental import pallas as pl


def kernel(features, w0, w1):
    raise NotImplementedError("write your pallas kernel here")


# speedup vs baseline: 12.6825x; 12.6825x over previous
import functools

import jax
import jax.numpy as jnp
from jax.experimental import pallas as pl

N = 8192
D = 512
KK = 33
BM = 128
BITERS = 32


def _emb_body(f_ref, w0_ref, w1_ref, emb_ref):
    h = jnp.maximum(f_ref[...] * w0_ref[...], 0.0) * w1_ref[...]
    nrm = jnp.sqrt(jnp.sum(h * h, axis=1, keepdims=True))
    emb_ref[...] = h / jnp.maximum(nrm, 1e-12)


def _topk_body(emb_blk_ref, emb_all_ref, out_ref, *, kk, biters):
    bm = emb_blk_ref.shape[0]
    sim = jax.lax.dot_general(
        emb_blk_ref[...], emb_all_ref[...],
        (((1,), (1,)), ((), ())),
        preferred_element_type=jnp.float32,
    )

    def step(_, carry):
        lo, hi = carry
        mid = (lo + hi) * 0.5
        cnt = jnp.sum(jnp.where(sim >= mid, 1.0, 0.0), axis=1, keepdims=True)
        pred = cnt >= float(kk)
        return jnp.where(pred, mid, lo), jnp.where(pred, hi, mid)

    lo0 = jnp.full((bm, 1), -2.0, jnp.float32)
    hi0 = jnp.full((bm, 1), 2.0, jnp.float32)
    lo, _ = jax.lax.fori_loop(0, biters, step, (lo0, hi0))
    out_ref[...] = jnp.where(sim >= lo, jnp.maximum(sim, 0.0), 0.0)


def _build(n, d, bm, kk, biters, interpret=False):
    emb_call = pl.pallas_call(
        _emb_body,
        grid=(8,),
        in_specs=[
            pl.BlockSpec((n // 8, d), lambda i: (i, 0)),
            pl.BlockSpec((1, d), lambda i: (0, 0)),
            pl.BlockSpec((1, d), lambda i: (0, 0)),
        ],
        out_specs=pl.BlockSpec((n // 8, d), lambda i: (i, 0)),
        out_shape=jax.ShapeDtypeStruct((n, d), jnp.float32),
        interpret=interpret,
    )
    topk_call = pl.pallas_call(
        functools.partial(_topk_body, kk=kk, biters=biters),
        grid=(n // bm,),
        in_specs=[
            pl.BlockSpec((bm, d), lambda i: (i, 0)),
            pl.BlockSpec((n, d), lambda i: (0, 0)),
        ],
        out_specs=pl.BlockSpec((bm, n), lambda i: (i, 0)),
        out_shape=jax.ShapeDtypeStruct((n, n), jnp.float32),
        interpret=interpret,
    )
    return emb_call, topk_call


def kernel(features, w0, w1):
    n, d = features.shape
    emb_call, topk_call = _build(n, d, BM, KK, BITERS)
    emb = emb_call(features, w0.reshape(1, d), w1.reshape(1, d))
    return topk_call(emb, emb)
